# baseline (speedup 1.0000x reference)
import jax
import jax.numpy as jnp
from jax.experimental import pallas as pl

_E = 64
_TOP_K = 8
_TM = 2048


def _gate_kernel_t(x_ref, w_ref, b_ref, wout_ref, iout_ref):
    logits = jax.lax.dot_general(
        w_ref[...], x_ref[...],
        dimension_numbers=(((0,), (1,)), ((), ())),
        preferred_element_type=jnp.float32,
    )
    logits = logits + b_ref[...]
    m = jnp.max(logits, axis=0, keepdims=True)
    e = jnp.exp(logits - m)
    s = jnp.sum(e, axis=0, keepdims=True)
    scores = e / s
    idx = jax.lax.broadcasted_iota(jnp.int32, scores.shape, 0)
    cur = scores
    ws = []
    inds = []
    for _ in range(_TOP_K):
        mk = jnp.max(cur, axis=0, keepdims=True)
        is_max = cur == mk
        ik = jnp.min(jnp.where(is_max, idx, _E), axis=0, keepdims=True)
        ws.append(mk)
        inds.append(ik)
        cur = jnp.where(idx == ik, -1.0, cur)
    ws_t = jnp.concatenate(ws, axis=0)
    inds_t = jnp.concatenate(inds, axis=0)
    eye = (jax.lax.broadcasted_iota(jnp.int32, (_TOP_K, _TOP_K), 0)
           == jax.lax.broadcasted_iota(jnp.int32, (_TOP_K, _TOP_K), 1)
           ).astype(jnp.float32)
    wout_ref[...] = jax.lax.dot_general(
        ws_t, eye,
        dimension_numbers=(((0,), (0,)), ((), ())),
        preferred_element_type=jnp.float32,
    )
    iout_ref[...] = jax.lax.dot_general(
        inds_t.astype(jnp.float32), eye,
        dimension_numbers=(((0,), (0,)), ((), ())),
        preferred_element_type=jnp.float32,
    ).astype(jnp.int32)


@jax.jit
def kernel(x, W, b):
    B, S, D = x.shape
    T = B * S
    x2 = x.reshape(T, D)
    b2 = b.reshape(_E, 1)
    grid = (T // _TM,)
    weights, indices = pl.pallas_call(
        _gate_kernel_t,
        grid=grid,
        in_specs=[
            pl.BlockSpec((_TM, D), lambda i: (i, 0)),
            pl.BlockSpec((D, _E), lambda i: (0, 0)),
            pl.BlockSpec((_E, 1), lambda i: (0, 0)),
        ],
        out_specs=[
            pl.BlockSpec((_TM, _TOP_K), lambda i: (i, 0)),
            pl.BlockSpec((_TM, _TOP_K), lambda i: (i, 0)),
        ],
        out_shape=[
            jax.ShapeDtypeStruct((T, _TOP_K), jnp.float32),
            jax.ShapeDtypeStruct((T, _TOP_K), jnp.int32),
        ],
    )(x2, W, b2)
    return weights.reshape(B, S, _TOP_K), indices.reshape(B, S, _TOP_K)


# speedup vs baseline: 1.2362x; 1.2362x over previous
import jax
import jax.numpy as jnp
from jax.experimental import pallas as pl

_E = 64
_TOP_K = 8
_TM = 2048


def _gate_kernel_t(x_ref, w_ref, b_ref, out_ref):
    logits = jax.lax.dot_general(
        w_ref[...], x_ref[...],
        dimension_numbers=(((0,), (1,)), ((), ())),
        preferred_element_type=jnp.float32,
    )
    logits = logits + b_ref[...]
    m = jnp.max(logits, axis=0, keepdims=True)
    e = jnp.exp(logits - m)
    s = jnp.sum(e, axis=0, keepdims=True)
    scores = e / s
    idx = jax.lax.broadcasted_iota(jnp.int32, scores.shape, 0)
    cur = scores
    ws = []
    inds = []
    for _ in range(_TOP_K):
        mk = jnp.max(cur, axis=0, keepdims=True)
        is_max = cur == mk
        ik = jnp.min(jnp.where(is_max, idx, _E), axis=0, keepdims=True)
        ws.append(mk)
        inds.append(ik)
        cur = jnp.where(idx == ik, -1.0, cur)
    packed = jnp.concatenate(
        [jax.lax.bitcast_convert_type(w, jnp.int32) for w in ws] + inds,
        axis=0)
    out_ref[...] = packed


@jax.jit
def kernel(x, W, b):
    B, S, D = x.shape
    T = B * S
    x2 = x.reshape(T, D)
    b2 = b.reshape(_E, 1)
    grid = (T // _TM,)
    packed_t = pl.pallas_call(
        _gate_kernel_t,
        grid=grid,
        in_specs=[
            pl.BlockSpec((_TM, D), lambda i: (i, 0)),
            pl.BlockSpec((D, _E), lambda i: (0, 0)),
            pl.BlockSpec((_E, 1), lambda i: (0, 0)),
        ],
        out_specs=pl.BlockSpec((2 * _TOP_K, _TM), lambda i: (0, i)),
        out_shape=jax.ShapeDtypeStruct((2 * _TOP_K, T), jnp.int32),
    )(x2, W, b2)
    y = packed_t.T.reshape(B, S, 2 * _TOP_K)
    weights = jax.lax.bitcast_convert_type(y[..., :_TOP_K], jnp.float32)
    indices = y[..., _TOP_K:]
    return weights, indices


# speedup vs baseline: 1.3418x; 1.0855x over previous
import jax
import jax.numpy as jnp
from jax.experimental import pallas as pl

_E = 64
_TOP_K = 8
_TM = 2048


def _gate_kernel_t(x_ref, w_ref, b_ref, wout_ref, iout_ref):
    logits = jax.lax.dot_general(
        w_ref[...], x_ref[...],
        dimension_numbers=(((0,), (1,)), ((), ())),
        preferred_element_type=jnp.float32,
    )
    logits = logits + b_ref[...]
    m = jnp.max(logits, axis=0, keepdims=True)
    e = jnp.exp(logits - m)
    s = jnp.sum(e, axis=0, keepdims=True)
    scores = e / s
    idx = jax.lax.broadcasted_iota(jnp.int32, scores.shape, 0)
    cur = scores
    ws = []
    inds = []
    for _ in range(_TOP_K):
        mk = jnp.max(cur, axis=0, keepdims=True)
        is_max = cur == mk
        ik = jnp.min(jnp.where(is_max, idx, _E), axis=0, keepdims=True)
        ws.append(mk)
        inds.append(ik)
        cur = jnp.where(idx == ik, -1.0, cur)
    wout_ref[...] = jnp.concatenate(ws, axis=0)
    iout_ref[...] = jnp.concatenate(inds, axis=0)


@jax.jit
def kernel(x, W, b):
    B, S, D = x.shape
    T = B * S
    x2 = x.reshape(T, D)
    b2 = b.reshape(_E, 1)
    grid = (T // _TM,)
    weights_t, indices_t = pl.pallas_call(
        _gate_kernel_t,
        grid=grid,
        in_specs=[
            pl.BlockSpec((_TM, D), lambda i: (i, 0)),
            pl.BlockSpec((D, _E), lambda i: (0, 0)),
            pl.BlockSpec((_E, 1), lambda i: (0, 0)),
        ],
        out_specs=[
            pl.BlockSpec((_TOP_K, _TM), lambda i: (0, i)),
            pl.BlockSpec((_TOP_K, _TM), lambda i: (0, i)),
        ],
        out_shape=[
            jax.ShapeDtypeStruct((_TOP_K, T), jnp.float32),
            jax.ShapeDtypeStruct((_TOP_K, T), jnp.int32),
        ],
    )(x2, W, b2)
    weights = weights_t.T.reshape(B, S, _TOP_K)
    indices = indices_t.T.reshape(B, S, _TOP_K)
    return weights, indices
